# baseline (speedup 1.0000x reference)
import functools

import jax
import jax.numpy as jnp
from jax import lax
from jax.experimental import pallas as pl
from jax.experimental.pallas import tpu as pltpu
from jax.experimental.pallas import tpu_sc as plsc

_NC = 2
_NS = 16
_NW = _NC * _NS
_CHUNK = 128


@functools.lru_cache(maxsize=None)
def _make_gather(batch, row):
    b_per_w = batch // _NW
    n_chunks = b_per_w // _CHUNK
    mesh = plsc.VectorSubcoreMesh(core_axis_name="c", subcore_axis_name="s")

    @functools.partial(
        pl.kernel,
        mesh=mesh,
        compiler_params=pltpu.CompilerParams(use_tc_tiling_on_sc=False),
        out_type=jax.ShapeDtypeStruct((batch, row), jnp.float32),
        scratch_types=[
            pltpu.VMEM((n_chunks, _CHUNK), jnp.int32),
            pltpu.VMEM((b_per_w, row), jnp.float32),
            pltpu.SemaphoreType.DMA,
        ],
    )
    def gather_kernel(mem_hbm, idx_hbm, out_hbm, idx_v, rows_v, sem):
        wid = lax.axis_index("s") * _NC + lax.axis_index("c")
        pltpu.sync_copy(idx_hbm.at[pl.ds(wid * n_chunks, n_chunks)], idx_v)
        copies = [
            pltpu.async_copy(
                mem_hbm.at[idx_v.at[j]],
                rows_v.at[pl.ds(j * _CHUNK, _CHUNK)],
                sem,
            )
            for j in range(n_chunks)
        ]
        for c in copies:
            c.wait()
        pltpu.sync_copy(rows_v, out_hbm.at[pl.ds(wid * b_per_w, b_per_w)])

    return gather_kernel


def kernel(memory, indices):
    batch = indices.shape[0]
    idx2d = indices.reshape(batch // _CHUNK, _CHUNK)
    return _make_gather(batch, memory.shape[1])(memory, idx2d)


# speedup vs baseline: 4.8600x; 4.8600x over previous
import functools

import jax
import jax.numpy as jnp
from jax import lax
from jax.experimental import pallas as pl
from jax.experimental.pallas import tpu as pltpu
from jax.experimental.pallas import tpu_sc as plsc

_NC = 2
_NS = 16
_NW = _NC * _NS
_ROW = 67
_K = 16


@functools.lru_cache(maxsize=None)
def _make_gather(batch):
    b_per_w = batch // _NW
    n_groups = b_per_w // _K
    mesh = plsc.VectorSubcoreMesh(core_axis_name="c", subcore_axis_name="s")

    @functools.partial(
        pl.kernel,
        mesh=mesh,
        compiler_params=pltpu.CompilerParams(needs_layout_passes=False),
        out_type=jax.ShapeDtypeStruct((batch, _ROW), jnp.float32),
        scratch_types=[
            pltpu.VMEM((b_per_w,), jnp.int32),
            pltpu.VMEM((b_per_w, _ROW), jnp.float32),
            pltpu.SemaphoreType.DMA,
        ],
    )
    def gather_kernel(mem_hbm, idx_hbm, out_hbm, idx_v, rows_v, sem):
        wid = lax.axis_index("s") * _NC + lax.axis_index("c")
        base = wid * b_per_w
        pltpu.sync_copy(idx_hbm.at[pl.ds(base, b_per_w)], idx_v)

        def do_group(g):
            j0 = g * _K
            vec = idx_v[pl.ds(j0, _K)]
            copies = []
            for k in range(_K):
                r = vec[k]
                copies.append(pltpu.async_copy(
                    mem_hbm.at[pl.ds(r, 1)],
                    rows_v.at[pl.ds(j0 + k, 1)],
                    sem,
                ))
            for c in copies:
                c.wait()

        pl.loop(0, n_groups)(do_group)
        pltpu.sync_copy(rows_v, out_hbm.at[pl.ds(base, b_per_w)])

    return gather_kernel


def kernel(memory, indices):
    return _make_gather(indices.shape[0])(memory, indices)


# speedup vs baseline: 5.0709x; 1.0434x over previous
import functools

import jax
import jax.numpy as jnp
from jax import lax
from jax.experimental import pallas as pl
from jax.experimental.pallas import tpu as pltpu
from jax.experimental.pallas import tpu_sc as plsc

_NC = 2
_NS = 16
_NW = _NC * _NS
_ROW = 67
_K = 32


@functools.lru_cache(maxsize=None)
def _make_gather(batch):
    b_per_w = batch // _NW
    n_groups = b_per_w // _K
    mesh = plsc.VectorSubcoreMesh(core_axis_name="c", subcore_axis_name="s")

    @functools.partial(
        pl.kernel,
        mesh=mesh,
        compiler_params=pltpu.CompilerParams(needs_layout_passes=False),
        out_type=jax.ShapeDtypeStruct((batch, _ROW), jnp.float32),
        scratch_types=[
            pltpu.VMEM((b_per_w,), jnp.int32),
            pltpu.VMEM((b_per_w, _ROW), jnp.float32),
            pltpu.SemaphoreType.DMA,
        ],
    )
    def gather_kernel(mem_hbm, idx_hbm, out_hbm, idx_v, rows_v, sem):
        wid = lax.axis_index("s") * _NC + lax.axis_index("c")
        base = wid * b_per_w
        pltpu.sync_copy(idx_hbm.at[pl.ds(base, b_per_w)], idx_v)

        def drain(n):
            for _ in range(n):
                pltpu.make_async_copy(
                    mem_hbm.at[pl.ds(0, 1)], rows_v.at[pl.ds(0, 1)], sem
                ).wait()

        def do_group(g):
            j0 = g * _K
            for b in range(_K // 16):
                vec = idx_v[pl.ds(j0 + b * 16, 16)]
                for k in range(16):
                    pltpu.async_copy(
                        mem_hbm.at[pl.ds(vec[k], 1)],
                        rows_v.at[pl.ds(j0 + b * 16 + k, 1)],
                        sem,
                    )
            @pl.when(g > 0)
            def _():
                drain(_K)

        pl.loop(0, n_groups)(do_group)
        drain(_K)
        pltpu.sync_copy(rows_v, out_hbm.at[pl.ds(base, b_per_w)])

    return gather_kernel


def kernel(memory, indices):
    return _make_gather(indices.shape[0])(memory, indices)


# speedup vs baseline: 5.0710x; 1.0000x over previous
import functools

import jax
import jax.numpy as jnp
from jax import lax
from jax.experimental import pallas as pl
from jax.experimental.pallas import tpu as pltpu
from jax.experimental.pallas import tpu_sc as plsc

_NC = 2
_NS = 16
_NW = _NC * _NS
_ROW = 67
_K = 32


@functools.lru_cache(maxsize=None)
def _make_gather(batch):
    b_per_w = batch // _NW
    n_groups = b_per_w // _K
    mesh = plsc.VectorSubcoreMesh(core_axis_name="c", subcore_axis_name="s")

    @functools.partial(
        pl.kernel,
        mesh=mesh,
        compiler_params=pltpu.CompilerParams(needs_layout_passes=False),
        out_type=jax.ShapeDtypeStruct((batch, _ROW), jnp.float32),
        scratch_types=[
            pltpu.VMEM((b_per_w,), jnp.int32),
            pltpu.VMEM((b_per_w, _ROW), jnp.float32),
            pltpu.SemaphoreType.DMA,
            pltpu.SemaphoreType.DMA,
            pltpu.SemaphoreType.DMA,
            pltpu.SemaphoreType.DMA,
        ],
    )
    def gather_kernel(mem_hbm, idx_hbm, out_hbm, idx_v, rows_v,
                      sem, sem1, sem2, sem3):
        sems = (sem, sem1, sem2, sem3)
        wid = lax.axis_index("s") * _NC + lax.axis_index("c")
        base = wid * b_per_w
        pltpu.sync_copy(idx_hbm.at[pl.ds(base, b_per_w)], idx_v)

        def drain(n):
            for i in range(n):
                pltpu.make_async_copy(
                    mem_hbm.at[pl.ds(0, 1)], rows_v.at[pl.ds(0, 1)],
                    sems[i % 4],
                ).wait()

        def do_group(g):
            j0 = g * _K
            for b in range(_K // 16):
                vec = idx_v[pl.ds(j0 + b * 16, 16)]
                for k in range(16):
                    pltpu.async_copy(
                        mem_hbm.at[pl.ds(vec[k], 1)],
                        rows_v.at[pl.ds(j0 + b * 16 + k, 1)],
                        sems[k % 4],
                    )
            @pl.when(g > 0)
            def _():
                drain(_K)

        pl.loop(0, n_groups)(do_group)
        drain(_K)
        pltpu.sync_copy(rows_v, out_hbm.at[pl.ds(base, b_per_w)])

    return gather_kernel


def kernel(memory, indices):
    return _make_gather(indices.shape[0])(memory, indices)
